# Initial kernel scaffold; baseline (speedup 1.0000x reference)
import jax
import jax.numpy as jnp
from jax.experimental import pallas as pl


def kernel(localizations, classifications, detection_threshold, localizations_default):
    raise NotImplementedError("write your pallas kernel here")


# speedup vs baseline: 29.3030x; 29.3030x over previous
import functools

import jax
import jax.numpy as jnp
from jax import lax
from jax.experimental import pallas as pl
from jax.experimental.pallas import tpu as pltpu
from jax.experimental.pallas import tpu_sc as plsc

N = 20000
PADN = N + 16
NRUNS = 16
OUTROWS = 208
TOPK = 200
OVERLAP = 0.45
CLS_THRESH = 0.01
VAR0, VAR1 = 0.1, 0.2
NEG = float("-inf")
SENT = -1e37
BIGI = 2**31 - 1
EPS = 1e-12


def _tc_body(cls_ref, loc_ref, dflt_ref, work_ref, lo_ref, hi_ref):
    c0 = cls_ref[0]
    c1 = cls_ref[1]
    c2 = cls_ref[2]
    c3 = cls_ref[3]
    c4 = cls_ref[4]
    m = jnp.maximum(jnp.maximum(jnp.maximum(c0, c1), jnp.maximum(c2, c3)), c4)
    e0 = jnp.exp(c0 - m)
    e1 = jnp.exp(c1 - m)
    e2 = jnp.exp(c2 - m)
    e3 = jnp.exp(c3 - m)
    e4 = jnp.exp(c4 - m)
    s = e0 + e1 + e2 + e3 + e4
    for k, ek in enumerate((e1, e2, e3, e4)):
        p = ek / s
        work_ref[pl.ds(4 * k, 4), :] = jnp.where(p > CLS_THRESH, p, NEG)
    l0 = loc_ref[0]
    l1 = loc_ref[1]
    d0 = dflt_ref[0][None, :]
    d1 = dflt_ref[1][None, :]
    cx = d0 + l0 * VAR0 * d1
    w = d1 * jnp.exp(l1 * VAR1)
    lo_ref[...] = cx - w / 2.0
    hi_ref[...] = cx + w / 2.0


def _tc_stage(cls_t, loc_t, dflt_t):
    return pl.pallas_call(
        _tc_body,
        out_shape=(
            jax.ShapeDtypeStruct((NRUNS, N), jnp.float32),
            jax.ShapeDtypeStruct((4, N), jnp.float32),
            jax.ShapeDtypeStruct((4, N), jnp.float32),
        ),
    )(cls_t, loc_t, dflt_t)


def _fused_pass(work_v, lo_v, hi_v, gidx_v, m_count, blo, bhi, blen, jg):
    lane = lax.iota(jnp.int32, 16)
    nch = (m_count + 15) // 16

    def chunk(i, carry):
        woffv, am, ai, ap = carry
        base = i * 16
        w = work_v[pl.ds(base, 16)]
        l = lo_v[pl.ds(base, 16)]
        h = hi_v[pl.ds(base, 16)]
        g = gidx_v[pl.ds(base, 16)]
        ln = jnp.maximum(h - l, 0.0)
        inter = jnp.maximum(jnp.minimum(h, bhi) - jnp.maximum(l, blo), 0.0)
        union = ln + blen - inter
        iou = inter / jnp.maximum(union, EPS)
        sup = (iou > OVERLAP) | (g == jg) | (w < SENT)
        keep = jnp.logical_not(sup)
        cs = plsc.cumsum(keep.astype(jnp.int32))
        pos = woffv + cs - 1
        plsc.store_scatter(work_v, [pos], w, mask=keep)
        plsc.store_scatter(lo_v, [pos], l, mask=keep)
        plsc.store_scatter(hi_v, [pos], h, mask=keep)
        plsc.store_scatter(gidx_v, [pos], g, mask=keep)
        wk = jnp.where(keep, w, NEG)
        gt = wk > am
        am = jnp.where(gt, wk, am)
        ai = jnp.where(gt, g, ai)
        ap = jnp.where(gt, pos, ap)
        woffv = woffv + plsc.all_reduce_population_count(keep)
        return woffv, am, ai, ap

    init = (
        jnp.zeros((16,), jnp.int32),
        jnp.full((16,), NEG, jnp.float32),
        jnp.full((16,), BIGI, jnp.int32),
        jnp.zeros((16,), jnp.int32),
    )
    woffv, am, ai, ap = lax.fori_loop(0, nch, chunk, init)
    new_count = jnp.max(woffv)
    plsc.store_scatter(
        work_v, [jnp.full((16,), new_count, jnp.int32) + lane],
        jnp.full((16,), NEG, jnp.float32))
    return new_count, am, ai, ap


def _sc_body(work_hbm, lo_hbm, hi_hbm, gidx_hbm, out_hbm,
             work_v, lo_v, hi_v, gidx_v, out_v):
    c = lax.axis_index("c")
    s = lax.axis_index("s")
    r = c * 8 + s
    lane = lax.iota(jnp.int32, 16)

    @pl.when(s < 8)
    def _run():
        b = r % 4
        pltpu.sync_copy(work_hbm.at[pl.ds(r * N, N)], work_v.at[pl.ds(0, N)])
        pltpu.sync_copy(lo_hbm.at[pl.ds(b * N, N)], lo_v.at[pl.ds(0, N)])
        pltpu.sync_copy(hi_hbm.at[pl.ds(b * N, N)], hi_v.at[pl.ds(0, N)])
        pltpu.sync_copy(gidx_hbm, gidx_v.at[pl.ds(0, N)])
        work_v[pl.ds(N, 16)] = jnp.full((16,), NEG, jnp.float32)

        far = jnp.full((16,), 2e30, jnp.float32)
        zero = jnp.zeros((16,), jnp.float32)
        state0 = _fused_pass(work_v, lo_v, hi_v, gidx_v,
                             jnp.int32(N), far, far, zero, jnp.int32(-1))

        def step(t, carry):
            m_count, am, ai, ap = carry
            m = jnp.max(am)
            valid = m > 0.0

            def do_step(_):
                elig = am == m
                jg = jnp.min(jnp.where(elig, ai, BIGI))
                sel = elig & (ai == jg)
                p = jnp.min(jnp.where(sel, ap, BIGI))
                pv = jnp.full((16,), p, jnp.int32)
                blo = plsc.load_gather(lo_v, [pv])
                bhi = plsc.load_gather(hi_v, [pv])
                sv = jnp.full((16,), m, jnp.float32)
                outvec = jnp.where(lane == 0, blo,
                                   jnp.where(lane == 1, bhi,
                                             jnp.where(lane == 2, sv, 0.0)))
                plsc.store_scatter(out_v, [t * 16 + lane], outvec)
                blen = jnp.maximum(bhi - blo, 0.0)
                return _fused_pass(work_v, lo_v, hi_v, gidx_v,
                                   m_count, blo, bhi, blen, jg)

            def no_step(_):
                plsc.store_scatter(out_v, [t * 16 + lane],
                                   jnp.zeros((16,), jnp.float32))
                return m_count, am, ai, ap

            return lax.cond(valid, do_step, no_step, None)

        lax.fori_loop(0, TOPK, step, state0)
        pltpu.sync_copy(out_v, out_hbm.at[pl.ds(r * OUTROWS * 16, OUTROWS * 16)])


@functools.partial(
    pl.kernel,
    out_type=jax.ShapeDtypeStruct((NRUNS * OUTROWS * 16,), jnp.float32),
    mesh=plsc.VectorSubcoreMesh(core_axis_name="c", subcore_axis_name="s"),
    scratch_types=[
        pltpu.VMEM((PADN,), jnp.float32),
        pltpu.VMEM((PADN,), jnp.float32),
        pltpu.VMEM((PADN,), jnp.float32),
        pltpu.VMEM((PADN,), jnp.int32),
        pltpu.VMEM((OUTROWS * 16,), jnp.float32),
    ],
    compiler_params=pltpu.CompilerParams(needs_layout_passes=False),
)
def _sc_nms(work_hbm, lo_hbm, hi_hbm, gidx_hbm, out_hbm,
            work_v, lo_v, hi_v, gidx_v, out_v):
    _sc_body(work_hbm, lo_hbm, hi_hbm, gidx_hbm, out_hbm,
             work_v, lo_v, hi_v, gidx_v, out_v)


@jax.jit
def kernel(localizations, classifications, detection_threshold,
           localizations_default):
    del detection_threshold
    cls_t = jnp.transpose(classifications, (2, 0, 1))
    loc_t = jnp.transpose(localizations, (2, 0, 1))
    dflt_t = jnp.transpose(localizations_default, (1, 0))
    work0, lo, hi = _tc_stage(cls_t, loc_t, dflt_t)
    gidx = jnp.arange(N, dtype=jnp.int32)
    out_flat = _sc_nms(work0.reshape(-1), lo.reshape(-1), hi.reshape(-1), gidx)
    out = out_flat.reshape(4, 4, OUTROWS, 16)[:, :, :TOPK, :3]
    return jnp.transpose(out, (1, 0, 2, 3))


# speedup vs baseline: 127.0432x; 4.3355x over previous
import functools

import jax
import jax.numpy as jnp
from jax import lax
from jax.experimental import pallas as pl
from jax.experimental.pallas import tpu as pltpu
from jax.experimental.pallas import tpu_sc as plsc

N = 20000
PADN = N + 16
NRUNS = 16
OUTROWS = 208
TOPK = 200
OVERLAP = 0.45
CLS_THRESH = 0.01
VAR0, VAR1 = 0.1, 0.2
NEG = float("-inf")
SENT = -1e37
BIGI = 2**31 - 1
EPS = 1e-12


def _tc_body(cls_ref, loc_ref, dflt_ref, work_ref, lo_ref, hi_ref):
    c0 = cls_ref[0]
    c1 = cls_ref[1]
    c2 = cls_ref[2]
    c3 = cls_ref[3]
    c4 = cls_ref[4]
    m = jnp.maximum(jnp.maximum(jnp.maximum(c0, c1), jnp.maximum(c2, c3)), c4)
    e0 = jnp.exp(c0 - m)
    e1 = jnp.exp(c1 - m)
    e2 = jnp.exp(c2 - m)
    e3 = jnp.exp(c3 - m)
    e4 = jnp.exp(c4 - m)
    s = e0 + e1 + e2 + e3 + e4
    for k, ek in enumerate((e1, e2, e3, e4)):
        p = ek / s
        work_ref[pl.ds(4 * k, 4), :] = jnp.where(p > CLS_THRESH, p, NEG)
    l0 = loc_ref[0]
    l1 = loc_ref[1]
    d0 = dflt_ref[0][None, :]
    d1 = dflt_ref[1][None, :]
    cx = d0 + l0 * VAR0 * d1
    w = d1 * jnp.exp(l1 * VAR1)
    lo_ref[...] = cx - w / 2.0
    hi_ref[...] = cx + w / 2.0


def _tc_stage(cls_t, loc_t, dflt_t):
    return pl.pallas_call(
        _tc_body,
        out_shape=(
            jax.ShapeDtypeStruct((NRUNS, N), jnp.float32),
            jax.ShapeDtypeStruct((4, N), jnp.float32),
            jax.ShapeDtypeStruct((4, N), jnp.float32),
        ),
    )(cls_t, loc_t, dflt_t)


def _pp_pass(src, dst, m_count, blo, bhi, blen):
    wsrc, lsrc, hsrc = src
    wdst, ldst, hdst = dst
    lane = lax.iota(jnp.int32, 16)
    nch = (m_count + 15) // 16
    init = (
        jnp.zeros((16,), jnp.int32),
        jnp.full((16,), NEG, jnp.float32),
        jnp.zeros((16,), jnp.int32),
    )

    def chunk(i, carry):
        woffv, am, ap = carry
        base = i * 16
        w = wsrc[pl.ds(base, 16)]
        l = lsrc[pl.ds(base, 16)]
        h = hsrc[pl.ds(base, 16)]
        ln = jnp.maximum(h - l, 0.0)
        inter = jnp.maximum(jnp.minimum(h, bhi) - jnp.maximum(l, blo), 0.0)
        union = ln + blen - inter
        iou = inter / jnp.maximum(union, EPS)
        sup = (iou > OVERLAP) | (w < SENT)
        keep = jnp.logical_not(sup)
        cs = plsc.cumsum(keep.astype(jnp.int32))
        pos = woffv + cs - 1
        plsc.store_scatter(wdst, [pos], w, mask=keep)
        plsc.store_scatter(ldst, [pos], l, mask=keep)
        plsc.store_scatter(hdst, [pos], h, mask=keep)
        wk = jnp.where(keep, w, NEG)
        gt = wk > am
        am = jnp.where(gt, wk, am)
        ap = jnp.where(gt, pos, ap)
        woffv = woffv + plsc.all_reduce_population_count(keep)
        return woffv, am, ap

    woffv, am, ap = plsc.parallel_loop(0, nch, unroll=4, carry=init)(chunk)
    new_count = jnp.max(woffv)
    plsc.store_scatter(
        wdst, [jnp.full((16,), new_count, jnp.int32) + lane],
        jnp.full((16,), NEG, jnp.float32))
    return new_count, am, ap


def _one_step(t, carry, src, dst, out_v, lane):
    m_count, am, ap = carry
    m = jnp.max(am)
    valid = m > 0.0
    wsrc, lsrc, hsrc = src

    def do_step(_):
        p = jnp.min(jnp.where(am == m, ap, BIGI))
        pv = jnp.full((16,), p, jnp.int32)
        blo = plsc.load_gather(lsrc, [pv])
        bhi = plsc.load_gather(hsrc, [pv])
        sv = jnp.full((16,), m, jnp.float32)
        outvec = jnp.where(lane == 0, blo,
                           jnp.where(lane == 1, bhi,
                                     jnp.where(lane == 2, sv, 0.0)))
        plsc.store_scatter(out_v, [t * 16 + lane], outvec)
        plsc.store_scatter(wsrc, [pv], jnp.full((16,), NEG, jnp.float32),
                           mask=lane == 0)
        blen = jnp.maximum(bhi - blo, 0.0)
        return _pp_pass(src, dst, m_count, blo, bhi, blen)

    def no_step(_):
        plsc.store_scatter(out_v, [t * 16 + lane],
                           jnp.zeros((16,), jnp.float32))
        return m_count, am, ap

    return lax.cond(valid, do_step, no_step, None)


def _sc_body(work_hbm, lo_hbm, hi_hbm, out_hbm,
             wa, la, ha, wb, lb, hb, out_v):
    c = lax.axis_index("c")
    s = lax.axis_index("s")
    r = c * 8 + s
    lane = lax.iota(jnp.int32, 16)

    @pl.when(s < 8)
    def _run():
        b = r % 4
        pltpu.sync_copy(work_hbm.at[pl.ds(r * N, N)], wa.at[pl.ds(0, N)])
        pltpu.sync_copy(lo_hbm.at[pl.ds(b * N, N)], la.at[pl.ds(0, N)])
        pltpu.sync_copy(hi_hbm.at[pl.ds(b * N, N)], ha.at[pl.ds(0, N)])

        aset = (wa, la, ha)
        bset = (wb, lb, hb)
        far = jnp.full((16,), 2e30, jnp.float32)
        zero = jnp.zeros((16,), jnp.float32)
        state0 = _pp_pass(aset, bset, jnp.int32(N), far, far, zero)

        def step_pair(t2, carry):
            carry = _one_step(2 * t2, carry, bset, aset, out_v, lane)
            carry = _one_step(2 * t2 + 1, carry, aset, bset, out_v, lane)
            return carry

        lax.fori_loop(0, TOPK // 2, step_pair, state0)
        pltpu.sync_copy(out_v, out_hbm.at[pl.ds(r * OUTROWS * 16, OUTROWS * 16)])


@functools.partial(
    pl.kernel,
    out_type=jax.ShapeDtypeStruct((NRUNS * OUTROWS * 16,), jnp.float32),
    mesh=plsc.VectorSubcoreMesh(core_axis_name="c", subcore_axis_name="s"),
    scratch_types=[
        pltpu.VMEM((PADN,), jnp.float32),
        pltpu.VMEM((PADN,), jnp.float32),
        pltpu.VMEM((PADN,), jnp.float32),
        pltpu.VMEM((PADN,), jnp.float32),
        pltpu.VMEM((PADN,), jnp.float32),
        pltpu.VMEM((PADN,), jnp.float32),
        pltpu.VMEM((OUTROWS * 16,), jnp.float32),
    ],
    compiler_params=pltpu.CompilerParams(needs_layout_passes=False),
)
def _sc_nms(work_hbm, lo_hbm, hi_hbm, out_hbm,
            wa, la, ha, wb, lb, hb, out_v):
    _sc_body(work_hbm, lo_hbm, hi_hbm, out_hbm,
             wa, la, ha, wb, lb, hb, out_v)


@jax.jit
def kernel(localizations, classifications, detection_threshold,
           localizations_default):
    del detection_threshold
    cls_t = jnp.transpose(classifications, (2, 0, 1))
    loc_t = jnp.transpose(localizations, (2, 0, 1))
    dflt_t = jnp.transpose(localizations_default, (1, 0))
    work0, lo, hi = _tc_stage(cls_t, loc_t, dflt_t)
    out_flat = _sc_nms(work0.reshape(-1), lo.reshape(-1), hi.reshape(-1))
    out = out_flat.reshape(4, 4, OUTROWS, 16)[:, :, :TOPK, :3]
    return jnp.transpose(out, (1, 0, 2, 3))


# speedup vs baseline: 132.7324x; 1.0448x over previous
import functools

import jax
import jax.numpy as jnp
from jax import lax
from jax.experimental import pallas as pl
from jax.experimental.pallas import tpu as pltpu
from jax.experimental.pallas import tpu_sc as plsc

N = 20000
PADN = N + 16
NRUNS = 16
OUTROWS = 208
TOPK = 200
OVERLAP = 0.45
CLS_THRESH = 0.01
VAR0, VAR1 = 0.1, 0.2
NEG = float("-inf")
SENT = -1e37
BIGI = 2**31 - 1
EPS = 1e-12


def _tc_body(cls_ref, loc_ref, dflt_ref, work_ref, lo_ref, hi_ref):
    c0 = cls_ref[0]
    c1 = cls_ref[1]
    c2 = cls_ref[2]
    c3 = cls_ref[3]
    c4 = cls_ref[4]
    m = jnp.maximum(jnp.maximum(jnp.maximum(c0, c1), jnp.maximum(c2, c3)), c4)
    e0 = jnp.exp(c0 - m)
    e1 = jnp.exp(c1 - m)
    e2 = jnp.exp(c2 - m)
    e3 = jnp.exp(c3 - m)
    e4 = jnp.exp(c4 - m)
    s = e0 + e1 + e2 + e3 + e4
    for k, ek in enumerate((e1, e2, e3, e4)):
        p = ek / s
        work_ref[pl.ds(4 * k, 4), :] = jnp.where(p > CLS_THRESH, p, NEG)
    l0 = loc_ref[0]
    l1 = loc_ref[1]
    d0 = dflt_ref[0][None, :]
    d1 = dflt_ref[1][None, :]
    cx = d0 + l0 * VAR0 * d1
    w = d1 * jnp.exp(l1 * VAR1)
    lo_ref[...] = cx - w / 2.0
    hi_ref[...] = cx + w / 2.0


def _tc_stage(cls_t, loc_t, dflt_t):
    return pl.pallas_call(
        _tc_body,
        out_shape=(
            jax.ShapeDtypeStruct((NRUNS, N), jnp.float32),
            jax.ShapeDtypeStruct((4, N), jnp.float32),
            jax.ShapeDtypeStruct((4, N), jnp.float32),
        ),
    )(cls_t, loc_t, dflt_t)


def _pp_pass(src, dst, m_count, blo, bhi, blen):
    wsrc, lsrc, hsrc = src
    wdst, ldst, hdst = dst
    lane = lax.iota(jnp.int32, 16)
    nch = (m_count + 15) // 16
    ones = jnp.full((16,), 1, jnp.int32)
    init = (
        jnp.full((16,), -1, jnp.int32),
        jnp.full((16,), NEG, jnp.float32),
        jnp.zeros((16,), jnp.int32),
    )

    def chunk(i, carry):
        woffv, am, ap = carry
        base = i * 16
        w = wsrc[pl.ds(base, 16)]
        l = lsrc[pl.ds(base, 16)]
        h = hsrc[pl.ds(base, 16)]
        ln = jnp.maximum(h - l, 0.0)
        inter = jnp.maximum(jnp.minimum(h, bhi) - jnp.maximum(l, blo), 0.0)
        union = ln + blen - inter
        iou = inter / jnp.maximum(union, EPS)
        keep = (iou <= OVERLAP) & (w > SENT)
        cs = plsc.cumsum(ones, mask=keep)
        pos = woffv + cs
        plsc.store_scatter(wdst, [pos], w, mask=keep)
        plsc.store_scatter(ldst, [pos], l, mask=keep)
        plsc.store_scatter(hdst, [pos], h, mask=keep)
        wk = jnp.where(keep, w, NEG)
        gt = wk > am
        am = jnp.where(gt, wk, am)
        ap = jnp.where(gt, pos, ap)
        woffv = woffv + plsc.all_reduce_population_count(keep)
        return woffv, am, ap

    woffv, am, ap = plsc.parallel_loop(0, nch, unroll=8, carry=init)(chunk)
    new_count = jnp.max(woffv) + 1
    plsc.store_scatter(
        wdst, [jnp.full((16,), new_count, jnp.int32) + lane],
        jnp.full((16,), NEG, jnp.float32))
    return new_count, am, ap


def _one_step(t, carry, src, dst, out_v, lane):
    m_count, am, ap = carry
    m = jnp.max(am)
    valid = m > 0.0
    wsrc, lsrc, hsrc = src

    def do_step(_):
        p = jnp.min(jnp.where(am == m, ap, BIGI))
        pv = jnp.full((16,), p, jnp.int32)
        blo = plsc.load_gather(lsrc, [pv])
        bhi = plsc.load_gather(hsrc, [pv])
        sv = jnp.full((16,), m, jnp.float32)
        outvec = jnp.where(lane == 0, blo,
                           jnp.where(lane == 1, bhi,
                                     jnp.where(lane == 2, sv, 0.0)))
        plsc.store_scatter(out_v, [t * 16 + lane], outvec)
        plsc.store_scatter(wsrc, [pv], jnp.full((16,), NEG, jnp.float32),
                           mask=lane == 0)
        blen = jnp.maximum(bhi - blo, 0.0)
        return _pp_pass(src, dst, m_count, blo, bhi, blen)

    def no_step(_):
        plsc.store_scatter(out_v, [t * 16 + lane],
                           jnp.zeros((16,), jnp.float32))
        return m_count, am, ap

    return lax.cond(valid, do_step, no_step, None)


def _sc_body(work_hbm, lo_hbm, hi_hbm, out_hbm,
             wa, la, ha, wb, lb, hb, out_v):
    c = lax.axis_index("c")
    s = lax.axis_index("s")
    r = c * 8 + s
    lane = lax.iota(jnp.int32, 16)

    @pl.when(s < 8)
    def _run():
        b = r % 4
        pltpu.sync_copy(work_hbm.at[pl.ds(r * N, N)], wa.at[pl.ds(0, N)])
        pltpu.sync_copy(lo_hbm.at[pl.ds(b * N, N)], la.at[pl.ds(0, N)])
        pltpu.sync_copy(hi_hbm.at[pl.ds(b * N, N)], ha.at[pl.ds(0, N)])

        aset = (wa, la, ha)
        bset = (wb, lb, hb)
        far = jnp.full((16,), 2e30, jnp.float32)
        zero = jnp.zeros((16,), jnp.float32)
        state0 = _pp_pass(aset, bset, jnp.int32(N), far, far, zero)

        def step_pair(t2, carry):
            carry = _one_step(2 * t2, carry, bset, aset, out_v, lane)
            carry = _one_step(2 * t2 + 1, carry, aset, bset, out_v, lane)
            return carry

        lax.fori_loop(0, TOPK // 2, step_pair, state0)
        pltpu.sync_copy(out_v, out_hbm.at[pl.ds(r * OUTROWS * 16, OUTROWS * 16)])


@functools.partial(
    pl.kernel,
    out_type=jax.ShapeDtypeStruct((NRUNS * OUTROWS * 16,), jnp.float32),
    mesh=plsc.VectorSubcoreMesh(core_axis_name="c", subcore_axis_name="s"),
    scratch_types=[
        pltpu.VMEM((PADN,), jnp.float32),
        pltpu.VMEM((PADN,), jnp.float32),
        pltpu.VMEM((PADN,), jnp.float32),
        pltpu.VMEM((PADN,), jnp.float32),
        pltpu.VMEM((PADN,), jnp.float32),
        pltpu.VMEM((PADN,), jnp.float32),
        pltpu.VMEM((OUTROWS * 16,), jnp.float32),
    ],
    compiler_params=pltpu.CompilerParams(needs_layout_passes=False),
)
def _sc_nms(work_hbm, lo_hbm, hi_hbm, out_hbm,
            wa, la, ha, wb, lb, hb, out_v):
    _sc_body(work_hbm, lo_hbm, hi_hbm, out_hbm,
             wa, la, ha, wb, lb, hb, out_v)


@jax.jit
def kernel(localizations, classifications, detection_threshold,
           localizations_default):
    del detection_threshold
    cls_t = jnp.transpose(classifications, (2, 0, 1))
    loc_t = jnp.transpose(localizations, (2, 0, 1))
    dflt_t = jnp.transpose(localizations_default, (1, 0))
    work0, lo, hi = _tc_stage(cls_t, loc_t, dflt_t)
    out_flat = _sc_nms(work0.reshape(-1), lo.reshape(-1), hi.reshape(-1))
    out = out_flat.reshape(4, 4, OUTROWS, 16)[:, :, :TOPK, :3]
    return jnp.transpose(out, (1, 0, 2, 3))


# speedup vs baseline: 137.4816x; 1.0358x over previous
import functools

import jax
import jax.numpy as jnp
from jax import lax
from jax.experimental import pallas as pl
from jax.experimental.pallas import tpu as pltpu
from jax.experimental.pallas import tpu_sc as plsc

N = 20000
PADN = N + 16
NRUNS = 16
OUTROWS = 208
TOPK = 200
OVERLAP = 0.45
CLS_THRESH = 0.01
VAR0, VAR1 = 0.1, 0.2
NEG = float("-inf")
SENT = -1e37
BIGI = 2**31 - 1
EPS = 1e-12


def _tc_body(cls_ref, loc_ref, dflt_ref, work_ref, lo_ref, hi_ref):
    c0 = cls_ref[0]
    c1 = cls_ref[1]
    c2 = cls_ref[2]
    c3 = cls_ref[3]
    c4 = cls_ref[4]
    m = jnp.maximum(jnp.maximum(jnp.maximum(c0, c1), jnp.maximum(c2, c3)), c4)
    e0 = jnp.exp(c0 - m)
    e1 = jnp.exp(c1 - m)
    e2 = jnp.exp(c2 - m)
    e3 = jnp.exp(c3 - m)
    e4 = jnp.exp(c4 - m)
    s = e0 + e1 + e2 + e3 + e4
    for k, ek in enumerate((e1, e2, e3, e4)):
        p = ek / s
        work_ref[pl.ds(4 * k, 4), :] = jnp.where(p > CLS_THRESH, p, NEG)
    l0 = loc_ref[0]
    l1 = loc_ref[1]
    d0 = dflt_ref[0][None, :]
    d1 = dflt_ref[1][None, :]
    cx = d0 + l0 * VAR0 * d1
    w = d1 * jnp.exp(l1 * VAR1)
    lo_ref[...] = cx - w / 2.0
    hi_ref[...] = cx + w / 2.0


def _tc_stage(cls_t, loc_t, dflt_t):
    return pl.pallas_call(
        _tc_body,
        out_shape=(
            jax.ShapeDtypeStruct((NRUNS, N), jnp.float32),
            jax.ShapeDtypeStruct((4, N), jnp.float32),
            jax.ShapeDtypeStruct((4, N), jnp.float32),
        ),
    )(cls_t, loc_t, dflt_t)


def _pp_pass(src, dst, m_count, box_a, box_b=None):
    wsrc, lsrc, hsrc = src
    wdst, ldst, hdst = dst
    lane = lax.iota(jnp.int32, 16)
    nch = (m_count + 15) // 16
    ones = jnp.full((16,), 1, jnp.int32)
    blo_a, bhi_a, blen_a = box_a
    init = (
        jnp.full((16,), -1, jnp.int32),
        jnp.full((16,), NEG, jnp.float32),
        jnp.zeros((16,), jnp.int32),
        jnp.full((16,), NEG, jnp.float32),
        jnp.zeros((16,), jnp.int32),
    )

    def chunk(i, carry):
        woffv, am1, ap1, am2, ap2 = carry
        base = i * 16
        w = wsrc[pl.ds(base, 16)]
        l = lsrc[pl.ds(base, 16)]
        h = hsrc[pl.ds(base, 16)]
        ln = jnp.maximum(h - l, 0.0)
        inter = jnp.maximum(jnp.minimum(h, bhi_a) - jnp.maximum(l, blo_a), 0.0)
        union = ln + blen_a - inter
        iou = inter / jnp.maximum(union, EPS)
        keep = (iou <= OVERLAP) & (w > SENT)
        if box_b is not None:
            blo_b, bhi_b, blen_b = box_b
            inter_b = jnp.maximum(
                jnp.minimum(h, bhi_b) - jnp.maximum(l, blo_b), 0.0)
            union_b = ln + blen_b - inter_b
            iou_b = inter_b / jnp.maximum(union_b, EPS)
            keep = keep & (iou_b <= OVERLAP)
        cs = plsc.cumsum(ones, mask=keep)
        pos = woffv + cs
        plsc.store_scatter(wdst, [pos], w, mask=keep)
        plsc.store_scatter(ldst, [pos], l, mask=keep)
        plsc.store_scatter(hdst, [pos], h, mask=keep)
        wk = jnp.where(keep, w, NEG)
        gt1 = wk > am1
        sv = jnp.where(gt1, am1, wk)
        sp = jnp.where(gt1, ap1, pos)
        am1 = jnp.where(gt1, wk, am1)
        ap1 = jnp.where(gt1, pos, ap1)
        gt2 = sv > am2
        am2 = jnp.where(gt2, sv, am2)
        ap2 = jnp.where(gt2, sp, ap2)
        woffv = woffv + plsc.all_reduce_population_count(keep)
        return woffv, am1, ap1, am2, ap2

    woffv, am1, ap1, am2, ap2 = plsc.parallel_loop(
        0, nch, unroll=8, carry=init)(chunk)
    new_count = jnp.max(woffv) + 1
    plsc.store_scatter(
        wdst, [jnp.full((16,), new_count, jnp.int32) + lane],
        jnp.full((16,), NEG, jnp.float32))
    return new_count, am1, ap1, am2, ap2


def _step_with(t, carry, src, dst, out_v, lane):
    m_count, am1, ap1, am2, ap2 = carry
    wsrc, lsrc, hsrc = src
    ma = jnp.max(am1)
    valid_a = ma > 0.0

    def do_step(_):
        pa = jnp.min(jnp.where(am1 == ma, ap1, BIGI))
        pav = jnp.full((16,), pa, jnp.int32)
        blo_a = plsc.load_gather(lsrc, [pav])
        bhi_a = plsc.load_gather(hsrc, [pav])
        sva = jnp.full((16,), ma, jnp.float32)
        out_a = jnp.where(lane == 0, blo_a,
                          jnp.where(lane == 1, bhi_a,
                                    jnp.where(lane == 2, sva, 0.0)))
        plsc.store_scatter(out_v, [t * 16 + lane], out_a)
        blen_a = jnp.maximum(bhi_a - blo_a, 0.0)
        on_a = ap1 == pa
        bm = jnp.where(on_a, am2, am1)
        bp = jnp.where(on_a, ap2, ap1)
        mb = jnp.max(bm)
        pb = jnp.min(jnp.where(bm == mb, bp, BIGI))
        pbv = jnp.full((16,), pb, jnp.int32)
        blo_b = plsc.load_gather(lsrc, [pbv])
        bhi_b = plsc.load_gather(hsrc, [pbv])
        ln_b = jnp.maximum(bhi_b - blo_b, 0.0)
        inter_ab = jnp.maximum(
            jnp.minimum(bhi_b, bhi_a) - jnp.maximum(blo_b, blo_a), 0.0)
        union_ab = ln_b + blen_a - inter_ab
        iou_ab = inter_ab / jnp.maximum(union_ab, EPS)
        pairv = (iou_ab <= OVERLAP).astype(jnp.int32)
        pair_ok = (mb > 0.0) & (jnp.max(pairv) > 0)

        def paired(_):
            svb = jnp.full((16,), mb, jnp.float32)
            out_b = jnp.where(lane == 0, blo_b,
                              jnp.where(lane == 1, bhi_b,
                                        jnp.where(lane == 2, svb, 0.0)))
            plsc.store_scatter(out_v, [(t + 1) * 16 + lane], out_b)
            kill = jnp.where(lane == 0, pa, pb)
            plsc.store_scatter(wsrc, [kill],
                               jnp.full((16,), NEG, jnp.float32),
                               mask=lane < 2)
            st = _pp_pass(src, dst, m_count,
                          (blo_a, bhi_a, blen_a), (blo_b, bhi_b, ln_b))
            return (jnp.int32(2),) + st

        def single(_):
            plsc.store_scatter(wsrc, [pav],
                               jnp.full((16,), NEG, jnp.float32),
                               mask=lane == 0)
            st = _pp_pass(src, dst, m_count, (blo_a, bhi_a, blen_a))
            return (jnp.int32(1),) + st

        return lax.cond(pair_ok, paired, single, None)

    def no_step(_):
        plsc.store_scatter(out_v, [t * 16 + lane],
                           jnp.zeros((16,), jnp.float32))
        return jnp.int32(1), m_count, am1, ap1, am2, ap2

    return lax.cond(valid_a, do_step, no_step, None)


def _sc_body(work_hbm, lo_hbm, hi_hbm, out_hbm,
             wa, la, ha, wb, lb, hb, out_v):
    c = lax.axis_index("c")
    s = lax.axis_index("s")
    r = c * 8 + s
    lane = lax.iota(jnp.int32, 16)

    @pl.when(s < 8)
    def _run():
        b = r % 4
        pltpu.sync_copy(work_hbm.at[pl.ds(r * N, N)], wa.at[pl.ds(0, N)])
        pltpu.sync_copy(lo_hbm.at[pl.ds(b * N, N)], la.at[pl.ds(0, N)])
        pltpu.sync_copy(hi_hbm.at[pl.ds(b * N, N)], ha.at[pl.ds(0, N)])

        aset = (wa, la, ha)
        bset = (wb, lb, hb)
        far = jnp.full((16,), 2e30, jnp.float32)
        zero = jnp.zeros((16,), jnp.float32)
        state0 = _pp_pass(aset, bset, jnp.int32(N), (far, far, zero))

        def wcond(carry):
            return carry[0] < TOPK

        def wbody(carry):
            t, parity, m_count, am1, ap1, am2, ap2 = carry
            st = (m_count, am1, ap1, am2, ap2)

            def even(_):
                return _step_with(t, st, bset, aset, out_v, lane)

            def odd(_):
                return _step_with(t, st, aset, bset, out_v, lane)

            adv, mc, b1, b2, b3, b4 = lax.cond(parity == 0, even, odd, None)
            return t + adv, 1 - parity, mc, b1, b2, b3, b4

        lax.while_loop(wcond, wbody,
                       (jnp.int32(0), jnp.int32(0)) + state0)
        pltpu.sync_copy(out_v, out_hbm.at[pl.ds(r * OUTROWS * 16, OUTROWS * 16)])


@functools.partial(
    pl.kernel,
    out_type=jax.ShapeDtypeStruct((NRUNS * OUTROWS * 16,), jnp.float32),
    mesh=plsc.VectorSubcoreMesh(core_axis_name="c", subcore_axis_name="s"),
    scratch_types=[
        pltpu.VMEM((PADN,), jnp.float32),
        pltpu.VMEM((PADN,), jnp.float32),
        pltpu.VMEM((PADN,), jnp.float32),
        pltpu.VMEM((PADN,), jnp.float32),
        pltpu.VMEM((PADN,), jnp.float32),
        pltpu.VMEM((PADN,), jnp.float32),
        pltpu.VMEM((OUTROWS * 16,), jnp.float32),
    ],
    compiler_params=pltpu.CompilerParams(needs_layout_passes=False),
)
def _sc_nms(work_hbm, lo_hbm, hi_hbm, out_hbm,
            wa, la, ha, wb, lb, hb, out_v):
    _sc_body(work_hbm, lo_hbm, hi_hbm, out_hbm,
             wa, la, ha, wb, lb, hb, out_v)


@jax.jit
def kernel(localizations, classifications, detection_threshold,
           localizations_default):
    del detection_threshold
    cls_t = jnp.transpose(classifications, (2, 0, 1))
    loc_t = jnp.transpose(localizations, (2, 0, 1))
    dflt_t = jnp.transpose(localizations_default, (1, 0))
    work0, lo, hi = _tc_stage(cls_t, loc_t, dflt_t)
    out_flat = _sc_nms(work0.reshape(-1), lo.reshape(-1), hi.reshape(-1))
    out = out_flat.reshape(4, 4, OUTROWS, 16)[:, :, :TOPK, :3]
    return jnp.transpose(out, (1, 0, 2, 3))


# speedup vs baseline: 143.2635x; 1.0421x over previous
import functools

import jax
import jax.numpy as jnp
from jax import lax
from jax.experimental import pallas as pl
from jax.experimental.pallas import tpu as pltpu
from jax.experimental.pallas import tpu_sc as plsc

N = 20000
PADN = N + 16
NRUNS = 16
OUTROWS = 208
TOPK = 200
OVERLAP = 0.45
CLS_THRESH = 0.01
VAR0, VAR1 = 0.1, 0.2
NEG = float("-inf")
SENT = -1e37
BIGI = 2**31 - 1
EPS = 1e-12


def _tc_body(cls_ref, loc_ref, dflt_ref, work_ref, lo_ref, hi_ref):
    c0 = cls_ref[0]
    c1 = cls_ref[1]
    c2 = cls_ref[2]
    c3 = cls_ref[3]
    c4 = cls_ref[4]
    m = jnp.maximum(jnp.maximum(jnp.maximum(c0, c1), jnp.maximum(c2, c3)), c4)
    e0 = jnp.exp(c0 - m)
    e1 = jnp.exp(c1 - m)
    e2 = jnp.exp(c2 - m)
    e3 = jnp.exp(c3 - m)
    e4 = jnp.exp(c4 - m)
    s = e0 + e1 + e2 + e3 + e4
    for k, ek in enumerate((e1, e2, e3, e4)):
        p = ek / s
        work_ref[pl.ds(4 * k, 4), :] = jnp.where(p > CLS_THRESH, p, NEG)
    l0 = loc_ref[0]
    l1 = loc_ref[1]
    d0 = dflt_ref[0][None, :]
    d1 = dflt_ref[1][None, :]
    cx = d0 + l0 * VAR0 * d1
    w = d1 * jnp.exp(l1 * VAR1)
    lo_ref[...] = cx - w / 2.0
    hi_ref[...] = cx + w / 2.0


def _tc_stage(cls_t, loc_t, dflt_t):
    return pl.pallas_call(
        _tc_body,
        out_shape=(
            jax.ShapeDtypeStruct((NRUNS, N), jnp.float32),
            jax.ShapeDtypeStruct((4, N), jnp.float32),
            jax.ShapeDtypeStruct((4, N), jnp.float32),
        ),
    )(cls_t, loc_t, dflt_t)


def _pp_pass(src, dst, m_count, box_a, box_b=None):
    wsrc, lsrc, hsrc = src
    wdst, ldst, hdst = dst
    lane = lax.iota(jnp.int32, 16)
    nch = (m_count + 15) // 16
    ones = jnp.full((16,), 1, jnp.int32)
    blo_a, bhi_a, blen_a = box_a
    init = (
        jnp.full((16,), -1, jnp.int32),
        jnp.full((16,), NEG, jnp.float32),
        jnp.zeros((16,), jnp.int32),
        jnp.full((16,), NEG, jnp.float32),
        jnp.zeros((16,), jnp.int32),
    )

    def chunk(i, carry):
        woffv, am1, ap1, am2, ap2 = carry
        base = i * 16
        w = wsrc[pl.ds(base, 16)]
        l = lsrc[pl.ds(base, 16)]
        h = hsrc[pl.ds(base, 16)]
        ln = jnp.maximum(h - l, 0.0)
        inter = jnp.maximum(jnp.minimum(h, bhi_a) - jnp.maximum(l, blo_a), 0.0)
        union = ln + blen_a - inter
        iou = inter / jnp.maximum(union, EPS)
        keep = (iou <= OVERLAP) & (w > SENT)
        if box_b is not None:
            blo_b, bhi_b, blen_b = box_b
            inter_b = jnp.maximum(
                jnp.minimum(h, bhi_b) - jnp.maximum(l, blo_b), 0.0)
            union_b = ln + blen_b - inter_b
            iou_b = inter_b / jnp.maximum(union_b, EPS)
            keep = keep & (iou_b <= OVERLAP)
        cs = plsc.cumsum(ones, mask=keep)
        pos = woffv + cs
        plsc.store_scatter(wdst, [pos], w, mask=keep)
        plsc.store_scatter(ldst, [pos], l, mask=keep)
        plsc.store_scatter(hdst, [pos], h, mask=keep)
        wk = jnp.where(keep, w, NEG)
        gt1 = wk > am1
        sv = jnp.where(gt1, am1, wk)
        sp = jnp.where(gt1, ap1, pos)
        am1 = jnp.where(gt1, wk, am1)
        ap1 = jnp.where(gt1, pos, ap1)
        gt2 = sv > am2
        am2 = jnp.where(gt2, sv, am2)
        ap2 = jnp.where(gt2, sp, ap2)
        woffv = woffv + plsc.all_reduce_population_count(keep)
        return woffv, am1, ap1, am2, ap2

    woffv, am1, ap1, am2, ap2 = plsc.parallel_loop(
        0, nch, unroll=16, carry=init)(chunk)
    new_count = jnp.max(woffv) + 1
    plsc.store_scatter(
        wdst, [jnp.full((16,), new_count, jnp.int32) + lane],
        jnp.full((16,), NEG, jnp.float32))
    return new_count, am1, ap1, am2, ap2


def _step_with(t, carry, src, dst, out_v, lane):
    m_count, am1, ap1, am2, ap2 = carry
    wsrc, lsrc, hsrc = src
    ma = jnp.max(am1)
    valid_a = ma > 0.0

    def do_step(_):
        pa = jnp.min(jnp.where(am1 == ma, ap1, BIGI))
        pav = jnp.full((16,), pa, jnp.int32)
        blo_a = plsc.load_gather(lsrc, [pav])
        bhi_a = plsc.load_gather(hsrc, [pav])
        sva = jnp.full((16,), ma, jnp.float32)
        out_a = jnp.where(lane == 0, blo_a,
                          jnp.where(lane == 1, bhi_a,
                                    jnp.where(lane == 2, sva, 0.0)))
        plsc.store_scatter(out_v, [t * 16 + lane], out_a)
        blen_a = jnp.maximum(bhi_a - blo_a, 0.0)
        on_a = ap1 == pa
        bm = jnp.where(on_a, am2, am1)
        bp = jnp.where(on_a, ap2, ap1)
        mb = jnp.max(bm)
        pb = jnp.min(jnp.where(bm == mb, bp, BIGI))
        pbv = jnp.full((16,), pb, jnp.int32)
        blo_b = plsc.load_gather(lsrc, [pbv])
        bhi_b = plsc.load_gather(hsrc, [pbv])
        ln_b = jnp.maximum(bhi_b - blo_b, 0.0)
        inter_ab = jnp.maximum(
            jnp.minimum(bhi_b, bhi_a) - jnp.maximum(blo_b, blo_a), 0.0)
        union_ab = ln_b + blen_a - inter_ab
        iou_ab = inter_ab / jnp.maximum(union_ab, EPS)
        pairv = (iou_ab <= OVERLAP).astype(jnp.int32)
        pair_ok = (mb > 0.0) & (jnp.max(pairv) > 0)

        def paired(_):
            svb = jnp.full((16,), mb, jnp.float32)
            out_b = jnp.where(lane == 0, blo_b,
                              jnp.where(lane == 1, bhi_b,
                                        jnp.where(lane == 2, svb, 0.0)))
            plsc.store_scatter(out_v, [(t + 1) * 16 + lane], out_b)
            kill = jnp.where(lane == 0, pa, pb)
            plsc.store_scatter(wsrc, [kill],
                               jnp.full((16,), NEG, jnp.float32),
                               mask=lane < 2)
            st = _pp_pass(src, dst, m_count,
                          (blo_a, bhi_a, blen_a), (blo_b, bhi_b, ln_b))
            return (jnp.int32(2),) + st

        def single(_):
            plsc.store_scatter(wsrc, [pav],
                               jnp.full((16,), NEG, jnp.float32),
                               mask=lane == 0)
            st = _pp_pass(src, dst, m_count, (blo_a, bhi_a, blen_a))
            return (jnp.int32(1),) + st

        return lax.cond(pair_ok, paired, single, None)

    def no_step(_):
        plsc.store_scatter(out_v, [t * 16 + lane],
                           jnp.zeros((16,), jnp.float32))
        return jnp.int32(1), m_count, am1, ap1, am2, ap2

    return lax.cond(valid_a, do_step, no_step, None)


def _sc_body(work_hbm, lo_hbm, hi_hbm, out_hbm,
             wa, la, ha, wb, lb, hb, out_v):
    c = lax.axis_index("c")
    s = lax.axis_index("s")
    r = c * 8 + s
    lane = lax.iota(jnp.int32, 16)

    @pl.when(s < 8)
    def _run():
        b = r % 4
        pltpu.sync_copy(work_hbm.at[pl.ds(r * N, N)], wa.at[pl.ds(0, N)])
        pltpu.sync_copy(lo_hbm.at[pl.ds(b * N, N)], la.at[pl.ds(0, N)])
        pltpu.sync_copy(hi_hbm.at[pl.ds(b * N, N)], ha.at[pl.ds(0, N)])

        aset = (wa, la, ha)
        bset = (wb, lb, hb)
        far = jnp.full((16,), 2e30, jnp.float32)
        zero = jnp.zeros((16,), jnp.float32)
        state0 = _pp_pass(aset, bset, jnp.int32(N), (far, far, zero))

        def wcond(carry):
            return carry[0] < TOPK

        def wbody(carry):
            t, parity, m_count, am1, ap1, am2, ap2 = carry
            st = (m_count, am1, ap1, am2, ap2)

            def even(_):
                return _step_with(t, st, bset, aset, out_v, lane)

            def odd(_):
                return _step_with(t, st, aset, bset, out_v, lane)

            adv, mc, b1, b2, b3, b4 = lax.cond(parity == 0, even, odd, None)
            return t + adv, 1 - parity, mc, b1, b2, b3, b4

        lax.while_loop(wcond, wbody,
                       (jnp.int32(0), jnp.int32(0)) + state0)
        pltpu.sync_copy(out_v, out_hbm.at[pl.ds(r * OUTROWS * 16, OUTROWS * 16)])


@functools.partial(
    pl.kernel,
    out_type=jax.ShapeDtypeStruct((NRUNS * OUTROWS * 16,), jnp.float32),
    mesh=plsc.VectorSubcoreMesh(core_axis_name="c", subcore_axis_name="s"),
    scratch_types=[
        pltpu.VMEM((PADN,), jnp.float32),
        pltpu.VMEM((PADN,), jnp.float32),
        pltpu.VMEM((PADN,), jnp.float32),
        pltpu.VMEM((PADN,), jnp.float32),
        pltpu.VMEM((PADN,), jnp.float32),
        pltpu.VMEM((PADN,), jnp.float32),
        pltpu.VMEM((OUTROWS * 16,), jnp.float32),
    ],
    compiler_params=pltpu.CompilerParams(needs_layout_passes=False),
)
def _sc_nms(work_hbm, lo_hbm, hi_hbm, out_hbm,
            wa, la, ha, wb, lb, hb, out_v):
    _sc_body(work_hbm, lo_hbm, hi_hbm, out_hbm,
             wa, la, ha, wb, lb, hb, out_v)


@jax.jit
def kernel(localizations, classifications, detection_threshold,
           localizations_default):
    del detection_threshold
    cls_t = jnp.transpose(classifications, (2, 0, 1))
    loc_t = jnp.transpose(localizations, (2, 0, 1))
    dflt_t = jnp.transpose(localizations_default, (1, 0))
    work0, lo, hi = _tc_stage(cls_t, loc_t, dflt_t)
    out_flat = _sc_nms(work0.reshape(-1), lo.reshape(-1), hi.reshape(-1))
    out = out_flat.reshape(4, 4, OUTROWS, 16)[:, :, :TOPK, :3]
    return jnp.transpose(out, (1, 0, 2, 3))
